# baseline (speedup 1.0000x reference)
import functools

import jax
import jax.numpy as jnp
from jax import lax
from jax.experimental import pallas as pl
from jax.experimental.pallas import tpu as pltpu
from jax.experimental.pallas import tpu_sc as plsc

_NQ = 64
_NX = 2 * _NQ
_K = 8192
_B = 262144
_NW = 32
_NS = 16
_CB = _B // _NW
_SB = 64
_NSB = _CB // _SB
_GPS = _SB // 16
_KPT = _K // _NS


def _traj_body(times_hbm, breaks_hbm, samples_hbm, out_hbm,
               pairs_sh, times_v, breaks_v, idxl_v, c_v, r_v,
               pb0, pb1, pb2, pb3,
               sem_g0, sem_g1, sem_g2, sem_g3,
               sem_w0, sem_w1, sem_w2, sem_w3):
    pbs = (pb0, pb1, pb2, pb3)
    sems_g = (sem_g0, sem_g1, sem_g2, sem_g3)
    sems_w = (sem_w0, sem_w1, sem_w2, sem_w3)

    nc = 2
    cid = lax.axis_index("c")
    sid = lax.axis_index("s")
    wid = sid * nc + cid
    base = wid * _CB

    r0 = sid * _KPT
    pltpu.sync_copy(samples_hbm.at[pl.ds(r0, _KPT)],
                    pairs_sh.at[pl.ds(r0, _KPT), pl.ds(0, _NQ)])

    @pl.when(sid < _NS - 1)
    def _():
        pltpu.sync_copy(samples_hbm.at[pl.ds(r0 + 1, _KPT)],
                        pairs_sh.at[pl.ds(r0, _KPT), pl.ds(_NQ, _NQ)])

    @pl.when(sid == _NS - 1)
    def _():
        pltpu.sync_copy(samples_hbm.at[pl.ds(r0 + 1, _KPT - 1)],
                        pairs_sh.at[pl.ds(r0, _KPT - 1), pl.ds(_NQ, _NQ)])

    pltpu.sync_copy(times_hbm.at[pl.ds(base, _CB)], times_v)
    pltpu.sync_copy(breaks_hbm, breaks_v)
    plsc.subcore_barrier()

    def search(s, b):
        slot = b * _SB

        @plsc.parallel_loop(0, _GPS, unroll=4)
        def search_group(gi):
            g = s * _GPS + gi
            t = times_v[pl.ds(g * 16, 16)]
            idxr = jnp.zeros((16,), jnp.int32)
            for sbit in (4096, 2048, 1024, 512, 256, 128, 64, 32, 16, 8,
                         4, 2, 1):
                probe = idxr + (sbit - 1)
                v = plsc.load_gather(breaks_v, [probe])
                idxr = jnp.where(v < t, idxr + sbit, idxr)
            idxl = jnp.maximum(idxr - 1, 0)
            bl = plsc.load_gather(breaks_v, [idxl])
            br = plsc.load_gather(breaks_v, [idxr])
            interp = jnp.clip((t - bl) / (br - bl), 0.0, 1.0)
            dt = jnp.maximum(br - bl, 1e-8)
            sl16 = pl.ds(slot + gi * 16, 16)
            idxl_v[sl16] = idxl
            c_v[sl16] = dt * interp
            r_v[sl16] = 1.0 / dt

    def gather(b):
        pltpu.async_copy(pairs_sh.at[idxl_v.at[pl.ds(b * _SB, _SB)]],
                         pbs[b], sems_g[b])

    def wait_gather(b):
        pltpu.make_async_copy(pairs_sh.at[idxl_v.at[pl.ds(0, _SB)]],
                              pbs[b], sems_g[b]).wait()

    def wait_write(b):
        pltpu.make_async_copy(pbs[b], out_hbm.at[pl.ds(base, _SB)],
                              sems_w[b]).wait()

    def compute_write(s, b):
        slot = b * _SB
        wait_gather(b)
        buf = pbs[b]

        @plsc.parallel_loop(0, _SB, unroll=8)
        def elem(e):
            esplat = jnp.full((16,), slot + e, jnp.int32)
            cv = plsc.load_gather(c_v, [esplat])
            rv = plsc.load_gather(r_v, [esplat])
            for j in range(4):
                slv = buf[e, pl.ds(j * 16, 16)]
                srv = buf[e, pl.ds(_NQ + j * 16, 16)]
                vel = (srv - slv) * rv
                buf[e, pl.ds(j * 16, 16)] = slv + vel * cv
                buf[e, pl.ds(_NQ + j * 16, 16)] = vel

        pltpu.async_copy(buf, out_hbm.at[pl.ds(base + s * _SB, _SB)],
                         sems_w[b])

    def outer(so, _):
        for b in range(4):
            s = so * 4 + b
            search(s, b)

            @pl.when(so > 0)
            def _():
                wait_write(b)

            gather(b)

            prev = (b - 1) % 4
            if b == 0:
                @pl.when(so > 0)
                def _():
                    compute_write(s - 1, prev)
            else:
                compute_write(s - 1, prev)
        return 0

    lax.fori_loop(0, _NSB // 4, outer, 0)
    compute_write(_NSB - 1, (_NSB - 1) % 4)
    for b in range(4):
        wait_write(b)


@functools.partial(
    pl.kernel,
    out_type=jax.ShapeDtypeStruct((_B, _NX), jnp.float32),
    mesh=plsc.VectorSubcoreMesh(core_axis_name="c", subcore_axis_name="s"),
    compiler_params=pltpu.CompilerParams(
        needs_layout_passes=False, use_tc_tiling_on_sc=False),
    scratch_types=[
        pltpu.VMEM_SHARED((_K, _NX), jnp.float32),
        pltpu.VMEM((_CB,), jnp.float32),
        pltpu.VMEM((_K,), jnp.float32),
        pltpu.VMEM((4 * _SB,), jnp.int32),
        pltpu.VMEM((4 * _SB,), jnp.float32),
        pltpu.VMEM((4 * _SB,), jnp.float32),
        pltpu.VMEM((_SB, _NX), jnp.float32),
        pltpu.VMEM((_SB, _NX), jnp.float32),
        pltpu.VMEM((_SB, _NX), jnp.float32),
        pltpu.VMEM((_SB, _NX), jnp.float32),
        pltpu.SemaphoreType.DMA,
        pltpu.SemaphoreType.DMA,
        pltpu.SemaphoreType.DMA,
        pltpu.SemaphoreType.DMA,
        pltpu.SemaphoreType.DMA,
        pltpu.SemaphoreType.DMA,
        pltpu.SemaphoreType.DMA,
        pltpu.SemaphoreType.DMA,
    ],
)
def _traj(times_hbm, breaks_hbm, samples_hbm, out_hbm, *scratch):
    _traj_body(times_hbm, breaks_hbm, samples_hbm, out_hbm, *scratch)


def kernel(times, breaks, samples):
    out = _traj(jnp.ravel(times), breaks, samples)
    return out.reshape(times.shape + (_NX,))


# speedup vs baseline: 1.2635x; 1.2635x over previous
import functools

import jax
import jax.numpy as jnp
from jax import lax
from jax.experimental import pallas as pl
from jax.experimental.pallas import tpu as pltpu
from jax.experimental.pallas import tpu_sc as plsc

_NQ = 64
_NX = 2 * _NQ
_K = 8192
_B = 262144
_NW = 32
_NS = 16
_CB = _B // _NW
_SB = 64
_NSB = _CB // _SB
_GPS = _SB // 16
_KPT = _K // _NS


def _traj_body(times_hbm, breaks_hbm, samples_hbm, out_hbm,
               pairs_sh, times_v, breaks_v, idxl_v, c_v, r_v,
               pb0, pb1, pb2, pb3,
               sem_g0, sem_g1, sem_g2, sem_g3,
               sem_w0, sem_w1, sem_w2, sem_w3):
    pbs = (pb0, pb1, pb2, pb3)
    sems_g = (sem_g0, sem_g1, sem_g2, sem_g3)
    sems_w = (sem_w0, sem_w1, sem_w2, sem_w3)

    nc = 2
    cid = lax.axis_index("c")
    sid = lax.axis_index("s")
    wid = sid * nc + cid
    base = wid * _CB

    r0 = sid * _KPT
    pltpu.async_copy(samples_hbm.at[pl.ds(r0, _KPT)],
                     pairs_sh.at[pl.ds(r0, _KPT), pl.ds(0, _NQ)], sem_g0)

    @pl.when(sid < _NS - 1)
    def _():
        pltpu.async_copy(samples_hbm.at[pl.ds(r0 + 1, _KPT)],
                         pairs_sh.at[pl.ds(r0, _KPT), pl.ds(_NQ, _NQ)],
                         sem_g1)

    @pl.when(sid == _NS - 1)
    def _():
        pltpu.async_copy(samples_hbm.at[pl.ds(r0 + 1, _KPT - 1)],
                         pairs_sh.at[pl.ds(r0, _KPT - 1), pl.ds(_NQ, _NQ)],
                         sem_g1)

    pltpu.async_copy(times_hbm.at[pl.ds(base, _CB)], times_v, sem_g2)
    pltpu.async_copy(breaks_hbm, breaks_v, sem_g3)

    pltpu.make_async_copy(samples_hbm.at[pl.ds(r0, _KPT)],
                          pairs_sh.at[pl.ds(r0, _KPT), pl.ds(0, _NQ)],
                          sem_g0).wait()

    @pl.when(sid < _NS - 1)
    def _():
        pltpu.make_async_copy(samples_hbm.at[pl.ds(r0 + 1, _KPT)],
                              pairs_sh.at[pl.ds(r0, _KPT), pl.ds(_NQ, _NQ)],
                              sem_g1).wait()

    @pl.when(sid == _NS - 1)
    def _():
        pltpu.make_async_copy(
            samples_hbm.at[pl.ds(r0 + 1, _KPT - 1)],
            pairs_sh.at[pl.ds(r0, _KPT - 1), pl.ds(_NQ, _NQ)],
            sem_g1).wait()

    pltpu.make_async_copy(times_hbm.at[pl.ds(base, _CB)], times_v,
                          sem_g2).wait()
    pltpu.make_async_copy(breaks_hbm, breaks_v, sem_g3).wait()
    plsc.subcore_barrier()

    def search(s, b):
        slot = b * _SB

        @plsc.parallel_loop(0, _GPS, unroll=2)
        def search_group(gi):
            g = s * _GPS + gi
            t = times_v[pl.ds(g * 16, 16)]
            idxr = jnp.zeros((16,), jnp.int32)
            for sbit in (4096, 2048, 1024, 512, 256, 128, 64, 32, 16, 8,
                         4, 2, 1):
                probe = idxr + (sbit - 1)
                v = plsc.load_gather(breaks_v, [probe])
                idxr = jnp.where(v < t, idxr + sbit, idxr)
            idxl = jnp.maximum(idxr - 1, 0)
            bl = plsc.load_gather(breaks_v, [idxl])
            br = plsc.load_gather(breaks_v, [idxr])
            interp = jnp.clip((t - bl) / (br - bl), 0.0, 1.0)
            dt = jnp.maximum(br - bl, 1e-8)
            sl16 = pl.ds(slot + gi * 16, 16)
            idxl_v[sl16] = idxl
            c_v[sl16] = dt * interp
            r_v[sl16] = 1.0 / dt

    def gather(b):
        pltpu.async_copy(pairs_sh.at[idxl_v.at[pl.ds(b * _SB, _SB)]],
                         pbs[b], sems_g[b])

    def wait_gather(b):
        pltpu.make_async_copy(pairs_sh.at[idxl_v.at[pl.ds(0, _SB)]],
                              pbs[b], sems_g[b]).wait()

    def wait_write(b):
        pltpu.make_async_copy(pbs[b], out_hbm.at[pl.ds(base, _SB)],
                              sems_w[b]).wait()

    def compute_write(s, b):
        slot = b * _SB
        wait_gather(b)
        buf = pbs[b]

        @plsc.parallel_loop(0, _SB, unroll=4)
        def elem(e):
            esplat = jnp.full((16,), slot + e, jnp.int32)
            cv = plsc.load_gather(c_v, [esplat])
            rv = plsc.load_gather(r_v, [esplat])
            for j in range(4):
                slv = buf[e, pl.ds(j * 16, 16)]
                srv = buf[e, pl.ds(_NQ + j * 16, 16)]
                vel = (srv - slv) * rv
                buf[e, pl.ds(j * 16, 16)] = slv + vel * cv
                buf[e, pl.ds(_NQ + j * 16, 16)] = vel

        pltpu.async_copy(buf, out_hbm.at[pl.ds(base + s * _SB, _SB)],
                         sems_w[b])

    def outer(so, _):
        for b in range(4):
            s = so * 4 + b
            search(s, b)

            @pl.when(so > 0)
            def _():
                wait_write(b)

            gather(b)

            prev = (b - 1) % 4
            if b == 0:
                @pl.when(so > 0)
                def _():
                    compute_write(s - 1, prev)
            else:
                compute_write(s - 1, prev)
        return 0

    lax.fori_loop(0, _NSB // 4, outer, 0)
    compute_write(_NSB - 1, (_NSB - 1) % 4)
    for b in range(4):
        wait_write(b)


@functools.partial(
    pl.kernel,
    out_type=jax.ShapeDtypeStruct((_B, _NX), jnp.float32),
    mesh=plsc.VectorSubcoreMesh(core_axis_name="c", subcore_axis_name="s"),
    compiler_params=pltpu.CompilerParams(
        needs_layout_passes=False, use_tc_tiling_on_sc=False),
    scratch_types=[
        pltpu.VMEM_SHARED((_K, _NX), jnp.float32),
        pltpu.VMEM((_CB,), jnp.float32),
        pltpu.VMEM((_K,), jnp.float32),
        pltpu.VMEM((4 * _SB,), jnp.int32),
        pltpu.VMEM((4 * _SB,), jnp.float32),
        pltpu.VMEM((4 * _SB,), jnp.float32),
        pltpu.VMEM((_SB, _NX), jnp.float32),
        pltpu.VMEM((_SB, _NX), jnp.float32),
        pltpu.VMEM((_SB, _NX), jnp.float32),
        pltpu.VMEM((_SB, _NX), jnp.float32),
        pltpu.SemaphoreType.DMA,
        pltpu.SemaphoreType.DMA,
        pltpu.SemaphoreType.DMA,
        pltpu.SemaphoreType.DMA,
        pltpu.SemaphoreType.DMA,
        pltpu.SemaphoreType.DMA,
        pltpu.SemaphoreType.DMA,
        pltpu.SemaphoreType.DMA,
    ],
)
def _traj(times_hbm, breaks_hbm, samples_hbm, out_hbm, *scratch):
    _traj_body(times_hbm, breaks_hbm, samples_hbm, out_hbm, *scratch)


def kernel(times, breaks, samples):
    out = _traj(jnp.ravel(times), breaks, samples)
    return out.reshape(times.shape + (_NX,))


# speedup vs baseline: 1.3439x; 1.0636x over previous
import functools

import jax
import jax.numpy as jnp
from jax import lax
from jax.experimental import pallas as pl
from jax.experimental.pallas import tpu as pltpu
from jax.experimental.pallas import tpu_sc as plsc

_NQ = 64
_NX = 2 * _NQ
_K = 8192
_B = 262144
_NW = 32
_NS = 16
_CB = _B // _NW
_SB = 64
_NSB = _CB // _SB
_GPS = _SB // 16
_KPT = _K // _NS


def _traj_body(times_hbm, breaks_hbm, samples_hbm, out_hbm,
               pairs_sh, times_v, breaks_v, idxl_v, c_v, r_v,
               pb0, pb1, pb2, pb3,
               sem_g0, sem_g1, sem_g2, sem_g3,
               sem_w0, sem_w1, sem_w2, sem_w3):
    pbs = (pb0, pb1, pb2, pb3)
    sems_g = (sem_g0, sem_g1, sem_g2, sem_g3)
    sems_w = (sem_w0, sem_w1, sem_w2, sem_w3)

    nc = 2
    cid = lax.axis_index("c")
    sid = lax.axis_index("s")
    wid = sid * nc + cid
    base = wid * _CB

    r0 = sid * _KPT
    pltpu.async_copy(samples_hbm.at[pl.ds(r0, _KPT)],
                     pairs_sh.at[pl.ds(r0, _KPT), pl.ds(0, _NQ)], sem_g0)

    @pl.when(sid < _NS - 1)
    def _():
        pltpu.async_copy(samples_hbm.at[pl.ds(r0 + 1, _KPT)],
                         pairs_sh.at[pl.ds(r0, _KPT), pl.ds(_NQ, _NQ)],
                         sem_g1)

    @pl.when(sid == _NS - 1)
    def _():
        pltpu.async_copy(samples_hbm.at[pl.ds(r0 + 1, _KPT - 1)],
                         pairs_sh.at[pl.ds(r0, _KPT - 1), pl.ds(_NQ, _NQ)],
                         sem_g1)

    pltpu.async_copy(times_hbm.at[pl.ds(base, _CB)], times_v, sem_g2)
    pltpu.async_copy(breaks_hbm, breaks_v, sem_g3)

    pltpu.make_async_copy(samples_hbm.at[pl.ds(r0, _KPT)],
                          pairs_sh.at[pl.ds(r0, _KPT), pl.ds(0, _NQ)],
                          sem_g0).wait()

    @pl.when(sid < _NS - 1)
    def _():
        pltpu.make_async_copy(samples_hbm.at[pl.ds(r0 + 1, _KPT)],
                              pairs_sh.at[pl.ds(r0, _KPT), pl.ds(_NQ, _NQ)],
                              sem_g1).wait()

    @pl.when(sid == _NS - 1)
    def _():
        pltpu.make_async_copy(
            samples_hbm.at[pl.ds(r0 + 1, _KPT - 1)],
            pairs_sh.at[pl.ds(r0, _KPT - 1), pl.ds(_NQ, _NQ)],
            sem_g1).wait()

    pltpu.make_async_copy(times_hbm.at[pl.ds(base, _CB)], times_v,
                          sem_g2).wait()
    pltpu.make_async_copy(breaks_hbm, breaks_v, sem_g3).wait()
    plsc.subcore_barrier()

    def search(s, b):
        slot = b * _SB

        @plsc.parallel_loop(0, _GPS, unroll=4)
        def search_group(gi):
            g = s * _GPS + gi
            t = times_v[pl.ds(g * 16, 16)]
            idxr = jnp.zeros((16,), jnp.int32)
            for sbit in (4096, 2048, 1024, 512, 256, 128, 64, 32, 16, 8,
                         4, 2, 1):
                probe = idxr + (sbit - 1)
                v = plsc.load_gather(breaks_v, [probe])
                idxr = jnp.where(v < t, idxr + sbit, idxr)
            idxl = jnp.maximum(idxr - 1, 0)
            bl = plsc.load_gather(breaks_v, [idxl])
            br = plsc.load_gather(breaks_v, [idxr])
            interp = jnp.clip((t - bl) / (br - bl), 0.0, 1.0)
            dt = jnp.maximum(br - bl, 1e-8)
            sl16 = pl.ds(slot + gi * 16, 16)
            idxl_v[sl16] = idxl
            c_v[sl16] = dt * interp
            r_v[sl16] = 1.0 / dt

    def gather(b):
        pltpu.async_copy(pairs_sh.at[idxl_v.at[pl.ds(b * _SB, _SB)]],
                         pbs[b], sems_g[b])

    def wait_gather(b):
        pltpu.make_async_copy(pairs_sh.at[idxl_v.at[pl.ds(0, _SB)]],
                              pbs[b], sems_g[b]).wait()

    def wait_write(b):
        pltpu.make_async_copy(pbs[b], out_hbm.at[pl.ds(base, _SB)],
                              sems_w[b]).wait()

    def compute_write(s, b):
        slot = b * _SB
        wait_gather(b)
        buf = pbs[b]

        @plsc.parallel_loop(0, _SB, unroll=4)
        def elem(e):
            esplat = jnp.full((16,), slot + e, jnp.int32)
            cv = plsc.load_gather(c_v, [esplat])
            rv = plsc.load_gather(r_v, [esplat])
            for j in range(4):
                slv = buf[e, pl.ds(j * 16, 16)]
                srv = buf[e, pl.ds(_NQ + j * 16, 16)]
                vel = (srv - slv) * rv
                buf[e, pl.ds(j * 16, 16)] = slv + vel * cv
                buf[e, pl.ds(_NQ + j * 16, 16)] = vel

        pltpu.async_copy(buf, out_hbm.at[pl.ds(base + s * _SB, _SB)],
                         sems_w[b])

    def outer(so, _):
        for b in range(4):
            s = so * 4 + b
            search(s, b)

            @pl.when(so > 0)
            def _():
                wait_write(b)

            gather(b)

            prev = (b - 1) % 4
            if b == 0:
                @pl.when(so > 0)
                def _():
                    compute_write(s - 1, prev)
            else:
                compute_write(s - 1, prev)
        return 0

    lax.fori_loop(0, _NSB // 4, outer, 0)
    compute_write(_NSB - 1, (_NSB - 1) % 4)
    for b in range(4):
        wait_write(b)


@functools.partial(
    pl.kernel,
    out_type=jax.ShapeDtypeStruct((_B, _NX), jnp.float32),
    mesh=plsc.VectorSubcoreMesh(core_axis_name="c", subcore_axis_name="s"),
    compiler_params=pltpu.CompilerParams(
        needs_layout_passes=False, use_tc_tiling_on_sc=False),
    scratch_types=[
        pltpu.VMEM_SHARED((_K, _NX), jnp.float32),
        pltpu.VMEM((_CB,), jnp.float32),
        pltpu.VMEM((_K,), jnp.float32),
        pltpu.VMEM((4 * _SB,), jnp.int32),
        pltpu.VMEM((4 * _SB,), jnp.float32),
        pltpu.VMEM((4 * _SB,), jnp.float32),
        pltpu.VMEM((_SB, _NX), jnp.float32),
        pltpu.VMEM((_SB, _NX), jnp.float32),
        pltpu.VMEM((_SB, _NX), jnp.float32),
        pltpu.VMEM((_SB, _NX), jnp.float32),
        pltpu.SemaphoreType.DMA,
        pltpu.SemaphoreType.DMA,
        pltpu.SemaphoreType.DMA,
        pltpu.SemaphoreType.DMA,
        pltpu.SemaphoreType.DMA,
        pltpu.SemaphoreType.DMA,
        pltpu.SemaphoreType.DMA,
        pltpu.SemaphoreType.DMA,
    ],
)
def _traj(times_hbm, breaks_hbm, samples_hbm, out_hbm, *scratch):
    _traj_body(times_hbm, breaks_hbm, samples_hbm, out_hbm, *scratch)


def kernel(times, breaks, samples):
    out = _traj(jnp.ravel(times), breaks, samples)
    return out.reshape(times.shape + (_NX,))
